# baseline (speedup 1.0000x reference)
import functools

import jax
import jax.numpy as jnp
from jax.experimental import pallas as pl

NUM_EMBEDDINGS = 1024
EMBEDDING_DIM = 64
COMMITMENT_COST = 0.25

N_ROWS = 32 * 576
BLOCK_R = 512
N_BLOCKS = N_ROWS // BLOCK_R


def _vq_body(x_ref, e_ref, ones_esq_ref, idx_ref, qst_ref, sse_ref, cnt_ref,
             loss_ref, ppl_ref):
    step = pl.program_id(0)
    x = x_ref[...]
    e = e_ref[...]
    sm2 = jax.lax.dot_general(-2.0 * x, e, (((1,), (1,)), ((), ())),
                              preferred_element_type=jnp.float32)
    xsq = jnp.sum(x * x, axis=1, keepdims=True)
    lhs2 = jnp.concatenate([xsq, jnp.ones_like(xsq)], axis=1)
    t = jax.lax.dot_general(lhs2, ones_esq_ref[...],
                            (((1,), (0,)), ((), ())),
                            preferred_element_type=jnp.float32)
    d = t + sm2

    m = jnp.min(d, axis=1, keepdims=True)
    iota = jax.lax.broadcasted_iota(jnp.int32, d.shape, 1)
    idx = jnp.min(jnp.where(d == m, iota, NUM_EMBEDDINGS),
                  axis=1, keepdims=True)
    idx_ref[...] = idx

    oh = (iota == idx).astype(jnp.float32)
    q = jax.lax.dot_general(oh, e, (((1,), (0,)), ((), ())),
                            preferred_element_type=jnp.float32)
    qst_ref[...] = q

    sse_part = jnp.sum((q - x) ** 2).reshape(1, 1)
    cnt_part = jnp.sum(oh, axis=0, keepdims=True)

    @pl.when(step == 0)
    def _init():
        sse_ref[...] = jnp.zeros_like(sse_ref)
        cnt_ref[...] = jnp.zeros_like(cnt_ref)

    sse_ref[...] += sse_part
    cnt_ref[...] += cnt_part

    @pl.when(step == N_BLOCKS - 1)
    def _finalize():
        mean_err = sse_ref[...] / (N_ROWS * EMBEDDING_DIM)
        loss_ref[...] = mean_err + COMMITMENT_COST * mean_err
        p = cnt_ref[...] / N_ROWS
        ent = jnp.sum(p * jnp.log(p + 1e-10)).reshape(1, 1)
        ppl_ref[...] = jnp.exp(-ent)


@functools.partial(jax.jit, static_argnames=("interpret",))
def _vq_call(flat_x, embedding, ones_esq, interpret=False):
    out_shapes = (
        jax.ShapeDtypeStruct((N_ROWS, 1), jnp.int32),
        jax.ShapeDtypeStruct((N_ROWS, EMBEDDING_DIM), jnp.float32),
        jax.ShapeDtypeStruct((1, 1), jnp.float32),
        jax.ShapeDtypeStruct((1, NUM_EMBEDDINGS), jnp.float32),
        jax.ShapeDtypeStruct((1, 1), jnp.float32),
        jax.ShapeDtypeStruct((1, 1), jnp.float32),
    )
    grid = (N_BLOCKS,)
    in_specs = [
        pl.BlockSpec((BLOCK_R, EMBEDDING_DIM), lambda i: (i, 0)),
        pl.BlockSpec((NUM_EMBEDDINGS, EMBEDDING_DIM), lambda i: (0, 0)),
        pl.BlockSpec((2, NUM_EMBEDDINGS), lambda i: (0, 0)),
    ]
    out_specs = (
        pl.BlockSpec((BLOCK_R, 1), lambda i: (i, 0)),
        pl.BlockSpec((BLOCK_R, EMBEDDING_DIM), lambda i: (i, 0)),
        pl.BlockSpec((1, 1), lambda i: (0, 0)),
        pl.BlockSpec((1, NUM_EMBEDDINGS), lambda i: (0, 0)),
        pl.BlockSpec((1, 1), lambda i: (0, 0)),
        pl.BlockSpec((1, 1), lambda i: (0, 0)),
    )
    return pl.pallas_call(
        _vq_body,
        grid=grid,
        in_specs=in_specs,
        out_specs=out_specs,
        out_shape=out_shapes,
        interpret=interpret,
    )(flat_x, embedding, ones_esq)


def kernel(inputs, embedding, interpret=False):
    input_shape = inputs.shape
    flat_x = inputs.reshape(-1, EMBEDDING_DIM)
    esq = jnp.sum(embedding ** 2, axis=1)[None, :]
    ones_esq = jnp.concatenate([jnp.ones_like(esq), esq], axis=0)
    idx, qst, _sse, _cnt, loss, ppl = _vq_call(flat_x, embedding, ones_esq,
                                               interpret=interpret)
    return (loss.reshape(()), qst.reshape(input_shape), ppl.reshape(()),
            idx.reshape(input_shape[:-1]))


# speedup vs baseline: 1.0556x; 1.0556x over previous
import functools

import jax
import jax.numpy as jnp
from jax.experimental import pallas as pl

NUM_EMBEDDINGS = 1024
EMBEDDING_DIM = 64
COMMITMENT_COST = 0.25

N_ROWS = 32 * 576
BLOCK_R = 512
N_BLOCKS = N_ROWS // BLOCK_R


def _vq_body(x_ref, e_ref, esq_ref, idx_ref, qst_ref, sse_ref, cnt_ref,
             loss_ref, ppl_ref):
    step = pl.program_id(0)
    x = x_ref[...]
    e = e_ref[...]
    sm2 = jax.lax.dot_general(-2.0 * x, e, (((1,), (1,)), ((), ())),
                              preferred_element_type=jnp.float32)
    xsq = jnp.sum(x * x, axis=1, keepdims=True)
    d = (xsq + esq_ref[...]) + sm2

    m = jnp.min(d, axis=1, keepdims=True)
    iota = jax.lax.broadcasted_iota(jnp.int32, d.shape, 1)
    idx = jnp.min(jnp.where(d == m, iota, NUM_EMBEDDINGS),
                  axis=1, keepdims=True)
    idx_ref[...] = idx

    oh = (iota == idx).astype(jnp.float32)
    q = jax.lax.dot_general(oh, e, (((1,), (0,)), ((), ())),
                            preferred_element_type=jnp.float32)
    qst_ref[...] = q

    sse_part = jnp.sum((q - x) ** 2).reshape(1, 1)
    cnt_part = jnp.sum(oh, axis=0, keepdims=True)

    @pl.when(step == 0)
    def _init():
        sse_ref[...] = jnp.zeros_like(sse_ref)
        cnt_ref[...] = jnp.zeros_like(cnt_ref)

    sse_ref[...] += sse_part
    cnt_ref[...] += cnt_part

    @pl.when(step == N_BLOCKS - 1)
    def _finalize():
        mean_err = sse_ref[...] / (N_ROWS * EMBEDDING_DIM)
        loss_ref[...] = mean_err + COMMITMENT_COST * mean_err
        p = cnt_ref[...] / N_ROWS
        ent = jnp.sum(p * jnp.log(p + 1e-10)).reshape(1, 1)
        ppl_ref[...] = jnp.exp(-ent)


@functools.partial(jax.jit, static_argnames=("interpret",))
def _vq_call(flat_x, embedding, esq, interpret=False):
    out_shapes = (
        jax.ShapeDtypeStruct((N_ROWS, 1), jnp.int32),
        jax.ShapeDtypeStruct((N_ROWS, EMBEDDING_DIM), jnp.float32),
        jax.ShapeDtypeStruct((1, 1), jnp.float32),
        jax.ShapeDtypeStruct((1, NUM_EMBEDDINGS), jnp.float32),
        jax.ShapeDtypeStruct((1, 1), jnp.float32),
        jax.ShapeDtypeStruct((1, 1), jnp.float32),
    )
    grid = (N_BLOCKS,)
    in_specs = [
        pl.BlockSpec((BLOCK_R, EMBEDDING_DIM), lambda i: (i, 0)),
        pl.BlockSpec((NUM_EMBEDDINGS, EMBEDDING_DIM), lambda i: (0, 0)),
        pl.BlockSpec((1, NUM_EMBEDDINGS), lambda i: (0, 0)),
    ]
    out_specs = (
        pl.BlockSpec((BLOCK_R, 1), lambda i: (i, 0)),
        pl.BlockSpec((BLOCK_R, EMBEDDING_DIM), lambda i: (i, 0)),
        pl.BlockSpec((1, 1), lambda i: (0, 0)),
        pl.BlockSpec((1, NUM_EMBEDDINGS), lambda i: (0, 0)),
        pl.BlockSpec((1, 1), lambda i: (0, 0)),
        pl.BlockSpec((1, 1), lambda i: (0, 0)),
    )
    return pl.pallas_call(
        _vq_body,
        grid=grid,
        in_specs=in_specs,
        out_specs=out_specs,
        out_shape=out_shapes,
        interpret=interpret,
    )(flat_x, embedding, esq)


def kernel(inputs, embedding, interpret=False):
    input_shape = inputs.shape
    flat_x = inputs.reshape(-1, EMBEDDING_DIM)
    esq = jnp.sum(embedding ** 2, axis=1)[None, :]
    idx, qst, _sse, _cnt, loss, ppl = _vq_call(flat_x, embedding, esq,
                                               interpret=interpret)
    return (loss.reshape(()), qst.reshape(input_shape), ppl.reshape(()),
            idx.reshape(input_shape[:-1]))


# speedup vs baseline: 1.2322x; 1.1673x over previous
import functools

import jax
import jax.numpy as jnp
from jax.experimental import pallas as pl

NUM_EMBEDDINGS = 1024
EMBEDDING_DIM = 64
COMMITMENT_COST = 0.25

N_ROWS = 32 * 576
BLOCK_R = 2304
N_BLOCKS = N_ROWS // BLOCK_R


def _vq_body(x_ref, e_ref, esq_ref, idx_ref, qst_ref, sse_ref, cnt_ref,
             loss_ref, ppl_ref):
    step = pl.program_id(0)
    x = x_ref[...]
    e = e_ref[...]
    sm2 = jax.lax.dot_general(-2.0 * x, e, (((1,), (1,)), ((), ())),
                              preferred_element_type=jnp.float32)
    xsq = jnp.sum(x * x, axis=1, keepdims=True)
    d = (xsq + esq_ref[...]) + sm2

    m = jnp.min(d, axis=1, keepdims=True)
    iota = jax.lax.broadcasted_iota(jnp.int32, d.shape, 1)
    idx = jnp.min(jnp.where(d == m, iota, NUM_EMBEDDINGS),
                  axis=1, keepdims=True)
    idx_ref[...] = idx

    oh = (iota == idx).astype(jnp.float32)
    q = jax.lax.dot_general(oh, e, (((1,), (0,)), ((), ())),
                            preferred_element_type=jnp.float32)
    qst_ref[...] = q

    sse_part = jnp.sum((q - x) ** 2).reshape(1, 1)
    cnt_part = jnp.sum(oh, axis=0, keepdims=True)

    @pl.when(step == 0)
    def _init():
        sse_ref[...] = jnp.zeros_like(sse_ref)
        cnt_ref[...] = jnp.zeros_like(cnt_ref)

    sse_ref[...] += sse_part
    cnt_ref[...] += cnt_part

    @pl.when(step == N_BLOCKS - 1)
    def _finalize():
        mean_err = sse_ref[...] / (N_ROWS * EMBEDDING_DIM)
        loss_ref[...] = mean_err + COMMITMENT_COST * mean_err
        p = cnt_ref[...] / N_ROWS
        ent = jnp.sum(p * jnp.log(p + 1e-10)).reshape(1, 1)
        ppl_ref[...] = jnp.exp(-ent)


@functools.partial(jax.jit, static_argnames=("interpret",))
def _vq_call(flat_x, embedding, esq, interpret=False):
    out_shapes = (
        jax.ShapeDtypeStruct((N_ROWS, 1), jnp.int32),
        jax.ShapeDtypeStruct((N_ROWS, EMBEDDING_DIM), jnp.float32),
        jax.ShapeDtypeStruct((1, 1), jnp.float32),
        jax.ShapeDtypeStruct((1, NUM_EMBEDDINGS), jnp.float32),
        jax.ShapeDtypeStruct((1, 1), jnp.float32),
        jax.ShapeDtypeStruct((1, 1), jnp.float32),
    )
    grid = (N_BLOCKS,)
    in_specs = [
        pl.BlockSpec((BLOCK_R, EMBEDDING_DIM), lambda i: (i, 0)),
        pl.BlockSpec((NUM_EMBEDDINGS, EMBEDDING_DIM), lambda i: (0, 0)),
        pl.BlockSpec((1, NUM_EMBEDDINGS), lambda i: (0, 0)),
    ]
    out_specs = (
        pl.BlockSpec((BLOCK_R, 1), lambda i: (i, 0)),
        pl.BlockSpec((BLOCK_R, EMBEDDING_DIM), lambda i: (i, 0)),
        pl.BlockSpec((1, 1), lambda i: (0, 0)),
        pl.BlockSpec((1, NUM_EMBEDDINGS), lambda i: (0, 0)),
        pl.BlockSpec((1, 1), lambda i: (0, 0)),
        pl.BlockSpec((1, 1), lambda i: (0, 0)),
    )
    return pl.pallas_call(
        _vq_body,
        grid=grid,
        in_specs=in_specs,
        out_specs=out_specs,
        out_shape=out_shapes,
        interpret=interpret,
    )(flat_x, embedding, esq)


def kernel(inputs, embedding, interpret=False):
    input_shape = inputs.shape
    flat_x = inputs.reshape(-1, EMBEDDING_DIM)
    esq = jnp.sum(embedding ** 2, axis=1)[None, :]
    idx, qst, _sse, _cnt, loss, ppl = _vq_call(flat_x, embedding, esq,
                                               interpret=interpret)
    return (loss.reshape(()), qst.reshape(input_shape), ppl.reshape(()),
            idx.reshape(input_shape[:-1]))
